# baseline (speedup 1.0000x reference)
import functools

import jax
import jax.numpy as jnp
from jax import lax
from jax.experimental import pallas as pl
from jax.experimental.pallas import tpu as pltpu
from jax.experimental.pallas import tpu_sc as plsc

B, C_IN, T = 16, 256, 2048
DVQ, K = 32, 8192
BLK = 512
NT = T // BLK
NCHUNKS = 2
BC = B // NCHUNKS

_PREC = lax.Precision.DEFAULT


def _e2_body(cb_ref, e2_ref):
    cb = cb_ref[...]
    e2 = jnp.sum(cb * cb, axis=1, keepdims=True)
    e2_ref[...] = jnp.broadcast_to(e2, (K, BLK))


def _encode_body(x_ref, w_in_ref, b_in_ref, cb_ref, e2_ref, idx_ref, z_ref):
    xb = x_ref[0]
    z = lax.dot_general(w_in_ref[...], xb,
                        (((1,), (0,)), ((), ())), precision=_PREC)
    z = z + b_in_ref[...]
    z_ref[0] = z
    cb = cb_ref[...]
    s2 = lax.dot_general(cb, z + z,
                         (((1,), (0,)), ((), ())), precision=_PREC)
    z2 = jnp.sum(z * z, axis=0, keepdims=True)
    dist = (z2 - s2) + e2_ref[...]
    idx = jnp.argmin(dist, axis=0).astype(jnp.int32)[None, None]
    idx_ref[...] = idx


def _decode_body(q_ref, w_out_ref, b_out_ref, mask_ref, z_ref,
                 out_ref, loss_ref):
    g = pl.program_id(0)
    qb = q_ref[0][:, :DVQ]
    out = lax.dot_general(w_out_ref[...], qb,
                          (((1,), (1,)), ((), ())), precision=_PREC)
    out_ref[0] = (out + b_out_ref[...]) * mask_ref[0]
    z = z_ref[0]
    cross = lax.dot_general(qb, z, (((0,), (1,)), ((), ())),
                            precision=lax.Precision.HIGHEST)
    eye = (lax.broadcasted_iota(jnp.int32, (DVQ, DVQ), 0)
           == lax.broadcasted_iota(jnp.int32, (DVQ, DVQ), 1))
    part = (jnp.sum(qb * qb) + jnp.sum(z * z)
            - 2.0 * jnp.sum(jnp.where(eye, cross, 0.0)))
    prev = jnp.where(g == 0, 0.0, loss_ref[0, 0])
    loss_ref[0, 0] = prev + part


_NC, _NS = 2, 16
_NW = _NC * _NS
_CHUNK = 128


@functools.cache
def _build_gather(rows):
    rows_w = rows // _NW
    nch = rows_w // _CHUNK
    mesh = plsc.VectorSubcoreMesh(core_axis_name="c", subcore_axis_name="s",
                                  num_cores=_NC, num_subcores=_NS)

    @functools.partial(
        pl.kernel,
        mesh=mesh,
        out_type=jax.ShapeDtypeStruct((rows, 128), jnp.float32),
        scratch_types=[
            pltpu.VMEM((nch, _CHUNK), jnp.int32),
            pltpu.VMEM((3, _CHUNK, 128), jnp.float32),
            pltpu.SemaphoreType.DMA,
            pltpu.SemaphoreType.DMA,
            pltpu.SemaphoreType.DMA,
            pltpu.SemaphoreType.DMA,
            pltpu.SemaphoreType.DMA,
            pltpu.SemaphoreType.DMA,
        ],
    )
    def _gather_body(cb_hbm, idx_hbm, q_hbm, idx_v, rows_v, *sems):
        wid = lax.axis_index("s") * _NC + lax.axis_index("c")
        gsem, ssem = sems[:3], sems[3:]
        pltpu.sync_copy(idx_hbm.at[pl.ds(wid * nch, nch)], idx_v)
        gathers = [
            pltpu.async_copy(cb_hbm.at[idx_v.at[j]], rows_v.at[j % 3],
                             gsem[j % 3])
            for j in range(min(2, nch))
        ]
        scatters = [None, None, None]
        for j in range(nch):
            gathers[j].wait()
            scatters[j % 3] = pltpu.async_copy(
                rows_v.at[j % 3],
                q_hbm.at[pl.ds(wid * rows_w + j * _CHUNK, _CHUNK)],
                ssem[j % 3])
            nxt = j + 2
            if nxt < nch:
                b = nxt % 3
                if scatters[b] is not None:
                    scatters[b].wait()
                    scatters[b] = None
                gathers.append(
                    pltpu.async_copy(cb_hbm.at[idx_v.at[nxt]],
                                     rows_v.at[b], gsem[b]))
        for s in scatters:
            if s is not None:
                s.wait()

    return _gather_body


def _encode(x, W_in, b_in, codebook, e2b):
    nb = x.shape[0]
    nstep = nb * T // BLK
    return pl.pallas_call(
        _encode_body,
        grid=(nstep,),
        in_specs=[
            pl.BlockSpec((1, C_IN, BLK), lambda g: (g // NT, 0, g % NT)),
            pl.BlockSpec((DVQ, C_IN), lambda g: (0, 0)),
            pl.BlockSpec((DVQ, 1), lambda g: (0, 0)),
            pl.BlockSpec((K, DVQ), lambda g: (0, 0)),
            pl.BlockSpec((K, BLK), lambda g: (0, 0)),
        ],
        out_specs=[
            pl.BlockSpec((1, 1, BLK), lambda g: (g, 0, 0)),
            pl.BlockSpec((1, DVQ, BLK), lambda g: (g, 0, 0)),
        ],
        out_shape=[
            jax.ShapeDtypeStruct((nstep, 1, BLK), jnp.int32),
            jax.ShapeDtypeStruct((nstep, DVQ, BLK), jnp.float32),
        ],
        compiler_params=pltpu.CompilerParams(
            dimension_semantics=("arbitrary",)),
    )(x, W_in, b_in.reshape(DVQ, 1), codebook, e2b)


def _decode(q, W_out, b_out, x_mask, z):
    nb = x_mask.shape[0]
    nstep = nb * T // BLK
    return pl.pallas_call(
        _decode_body,
        grid=(nstep,),
        in_specs=[
            pl.BlockSpec((1, BLK, 128), lambda g: (g, 0, 0)),
            pl.BlockSpec((C_IN, DVQ), lambda g: (0, 0)),
            pl.BlockSpec((C_IN, 1), lambda g: (0, 0)),
            pl.BlockSpec((1, 1, BLK), lambda g: (g // NT, 0, g % NT)),
            pl.BlockSpec((1, DVQ, BLK), lambda g: (g, 0, 0)),
        ],
        out_specs=[
            pl.BlockSpec((1, C_IN, BLK), lambda g: (g // NT, 0, g % NT)),
            pl.BlockSpec(memory_space=pltpu.SMEM, block_shape=(1, 1),
                         index_map=lambda g: (0, 0)),
        ],
        out_shape=[
            jax.ShapeDtypeStruct((nb, C_IN, T), jnp.float32),
            jax.ShapeDtypeStruct((1, 1), jnp.float32),
        ],
        compiler_params=pltpu.CompilerParams(
            dimension_semantics=("arbitrary",)),
    )(q.reshape(nstep, BLK, 128), W_out, b_out.reshape(C_IN, 1), x_mask, z)


def _gather_codebook(cb_pad, idx2):
    return _build_gather(idx2.size)(cb_pad, idx2)


def kernel(x, x_mask, W_in, b_in, codebook, W_out, b_out):
    e2b = pl.pallas_call(
        _e2_body,
        in_specs=[pl.BlockSpec((K, DVQ), lambda: (0, 0))],
        out_specs=pl.BlockSpec((K, BLK), lambda: (0, 0)),
        out_shape=jax.ShapeDtypeStruct((K, BLK), jnp.float32),
    )(codebook)
    cb_pad = jnp.pad(codebook, ((0, 0), (0, 128 - DVQ)))
    rows_c = BC * T
    idxs, qs, zs, outs, losses = [], [], [], [], []
    for c in range(NCHUNKS):
        xc = x[c * BC:(c + 1) * BC]
        idx2, zc = _encode(xc, W_in, b_in, codebook, e2b)
        idxs.append(idx2)
        zs.append(zc)
        qs.append(_gather_codebook(
            cb_pad, idx2.reshape(rows_c // _CHUNK, _CHUNK)))
    for c in range(NCHUNKS):
        outc, loss_sum = _decode(qs[c], W_out, b_out,
                                 x_mask[c * BC:(c + 1) * BC], zs[c])
        outs.append(outc)
        losses.append(loss_sum[0, 0])
    out = jnp.concatenate(outs, axis=0)
    indices = jnp.concatenate([i.reshape(BC, T) for i in idxs], axis=0)
    loss = sum(losses) / jnp.float32(B * T * DVQ)
    return (out, indices, loss)


# speedup vs baseline: 1.1489x; 1.1489x over previous
import functools

import jax
import jax.numpy as jnp
from jax import lax
from jax.experimental import pallas as pl
from jax.experimental.pallas import tpu as pltpu
from jax.experimental.pallas import tpu_sc as plsc

B, C_IN, T = 16, 256, 2048
DVQ, K = 32, 8192
BLK = 512
NT = T // BLK
NCHUNKS = 1
BC = B // NCHUNKS

_PREC = lax.Precision.DEFAULT


def _e2_body(cb_ref, e2_ref):
    cb = cb_ref[...]
    e2 = jnp.sum(cb * cb, axis=1, keepdims=True)
    e2_ref[...] = jnp.broadcast_to(e2, (K, BLK))


def _encode_body(x_ref, w_in_ref, b_in_ref, cb_ref, e2_ref, idx_ref, z_ref):
    xb = x_ref[0]
    z = lax.dot_general(w_in_ref[...], xb,
                        (((1,), (0,)), ((), ())), precision=_PREC)
    z = z + b_in_ref[...]
    z_ref[0] = z
    cb = cb_ref[...]
    s2 = lax.dot_general(cb, z + z,
                         (((1,), (0,)), ((), ())), precision=_PREC)
    z2 = jnp.sum(z * z, axis=0, keepdims=True)
    dist = (z2 - s2) + e2_ref[...]
    idx = jnp.argmin(dist, axis=0).astype(jnp.int32)[None, None]
    idx_ref[...] = idx


def _decode_body(q_ref, w_out_ref, b_out_ref, mask_ref, z_ref,
                 out_ref, loss_ref):
    g = pl.program_id(0)
    qb = q_ref[0][:, :DVQ]
    out = lax.dot_general(w_out_ref[...], qb,
                          (((1,), (1,)), ((), ())), precision=_PREC)
    out_ref[0] = (out + b_out_ref[...]) * mask_ref[0]
    z = z_ref[0]
    cross = lax.dot_general(qb, z, (((0,), (1,)), ((), ())),
                            precision=lax.Precision.HIGHEST)
    eye = (lax.broadcasted_iota(jnp.int32, (DVQ, DVQ), 0)
           == lax.broadcasted_iota(jnp.int32, (DVQ, DVQ), 1))
    part = (jnp.sum(qb * qb) + jnp.sum(z * z)
            - 2.0 * jnp.sum(jnp.where(eye, cross, 0.0)))
    prev = jnp.where(g == 0, 0.0, loss_ref[0, 0])
    loss_ref[0, 0] = prev + part


_NC, _NS = 2, 16
_NW = _NC * _NS
_CHUNK = 128


@functools.cache
def _build_gather(rows):
    rows_w = rows // _NW
    nch = rows_w // _CHUNK
    mesh = plsc.VectorSubcoreMesh(core_axis_name="c", subcore_axis_name="s",
                                  num_cores=_NC, num_subcores=_NS)

    @functools.partial(
        pl.kernel,
        mesh=mesh,
        out_type=jax.ShapeDtypeStruct((rows, 128), jnp.float32),
        scratch_types=[
            pltpu.VMEM((nch, _CHUNK), jnp.int32),
            pltpu.VMEM((3, _CHUNK, 128), jnp.float32),
            pltpu.SemaphoreType.DMA,
            pltpu.SemaphoreType.DMA,
            pltpu.SemaphoreType.DMA,
            pltpu.SemaphoreType.DMA,
            pltpu.SemaphoreType.DMA,
            pltpu.SemaphoreType.DMA,
        ],
    )
    def _gather_body(cb_hbm, idx_hbm, q_hbm, idx_v, rows_v, *sems):
        wid = lax.axis_index("s") * _NC + lax.axis_index("c")
        gsem, ssem = sems[:3], sems[3:]
        pltpu.sync_copy(idx_hbm.at[pl.ds(wid * nch, nch)], idx_v)
        gathers = [
            pltpu.async_copy(cb_hbm.at[idx_v.at[j]], rows_v.at[j % 3],
                             gsem[j % 3])
            for j in range(min(2, nch))
        ]
        scatters = [None, None, None]
        for j in range(nch):
            gathers[j].wait()
            scatters[j % 3] = pltpu.async_copy(
                rows_v.at[j % 3],
                q_hbm.at[pl.ds(wid * rows_w + j * _CHUNK, _CHUNK)],
                ssem[j % 3])
            nxt = j + 2
            if nxt < nch:
                b = nxt % 3
                if scatters[b] is not None:
                    scatters[b].wait()
                    scatters[b] = None
                gathers.append(
                    pltpu.async_copy(cb_hbm.at[idx_v.at[nxt]],
                                     rows_v.at[b], gsem[b]))
        for s in scatters:
            if s is not None:
                s.wait()

    return _gather_body


def _encode(x, W_in, b_in, codebook, e2b):
    nb = x.shape[0]
    nstep = nb * T // BLK
    return pl.pallas_call(
        _encode_body,
        grid=(nstep,),
        in_specs=[
            pl.BlockSpec((1, C_IN, BLK), lambda g: (g // NT, 0, g % NT)),
            pl.BlockSpec((DVQ, C_IN), lambda g: (0, 0)),
            pl.BlockSpec((DVQ, 1), lambda g: (0, 0)),
            pl.BlockSpec((K, DVQ), lambda g: (0, 0)),
            pl.BlockSpec((K, BLK), lambda g: (0, 0)),
        ],
        out_specs=[
            pl.BlockSpec((1, 1, BLK), lambda g: (g, 0, 0)),
            pl.BlockSpec((1, DVQ, BLK), lambda g: (g, 0, 0)),
        ],
        out_shape=[
            jax.ShapeDtypeStruct((nstep, 1, BLK), jnp.int32),
            jax.ShapeDtypeStruct((nstep, DVQ, BLK), jnp.float32),
        ],
        compiler_params=pltpu.CompilerParams(
            dimension_semantics=("arbitrary",)),
    )(x, W_in, b_in.reshape(DVQ, 1), codebook, e2b)


def _decode(q, W_out, b_out, x_mask, z):
    nb = x_mask.shape[0]
    nstep = nb * T // BLK
    return pl.pallas_call(
        _decode_body,
        grid=(nstep,),
        in_specs=[
            pl.BlockSpec((1, BLK, 128), lambda g: (g, 0, 0)),
            pl.BlockSpec((C_IN, DVQ), lambda g: (0, 0)),
            pl.BlockSpec((C_IN, 1), lambda g: (0, 0)),
            pl.BlockSpec((1, 1, BLK), lambda g: (g // NT, 0, g % NT)),
            pl.BlockSpec((1, DVQ, BLK), lambda g: (g, 0, 0)),
        ],
        out_specs=[
            pl.BlockSpec((1, C_IN, BLK), lambda g: (g // NT, 0, g % NT)),
            pl.BlockSpec(memory_space=pltpu.SMEM, block_shape=(1, 1),
                         index_map=lambda g: (0, 0)),
        ],
        out_shape=[
            jax.ShapeDtypeStruct((nb, C_IN, T), jnp.float32),
            jax.ShapeDtypeStruct((1, 1), jnp.float32),
        ],
        compiler_params=pltpu.CompilerParams(
            dimension_semantics=("arbitrary",)),
    )(q.reshape(nstep, BLK, 128), W_out, b_out.reshape(C_IN, 1), x_mask, z)


def _gather_codebook(cb_pad, idx2):
    return _build_gather(idx2.size)(cb_pad, idx2)


def kernel(x, x_mask, W_in, b_in, codebook, W_out, b_out):
    e2b = pl.pallas_call(
        _e2_body,
        in_specs=[pl.BlockSpec((K, DVQ), lambda: (0, 0))],
        out_specs=pl.BlockSpec((K, BLK), lambda: (0, 0)),
        out_shape=jax.ShapeDtypeStruct((K, BLK), jnp.float32),
    )(codebook)
    cb_pad = jnp.pad(codebook, ((0, 0), (0, 128 - DVQ)))
    rows_c = BC * T
    idxs, qs, zs, outs, losses = [], [], [], [], []
    for c in range(NCHUNKS):
        xc = x[c * BC:(c + 1) * BC]
        idx2, zc = _encode(xc, W_in, b_in, codebook, e2b)
        idxs.append(idx2)
        zs.append(zc)
        qs.append(_gather_codebook(
            cb_pad, idx2.reshape(rows_c // _CHUNK, _CHUNK)))
    for c in range(NCHUNKS):
        outc, loss_sum = _decode(qs[c], W_out, b_out,
                                 x_mask[c * BC:(c + 1) * BC], zs[c])
        outs.append(outc)
        losses.append(loss_sum[0, 0])
    out = jnp.concatenate(outs, axis=0)
    indices = jnp.concatenate([i.reshape(BC, T) for i in idxs], axis=0)
    loss = sum(losses) / jnp.float32(B * T * DVQ)
    return (out, indices, loss)
